# baseline (speedup 1.0000x reference)
import functools

import jax
import jax.numpy as jnp
from jax import lax
from jax.experimental import pallas as pl
from jax.experimental.pallas import tpu as pltpu
from jax.experimental.pallas import tpu_sc as plsc

HALF = 128
D = 256
NQ = 16384
NW = 32
VPW = NQ // NW
CH = 16
NCH = VPW // CH
ROWS = CH * 8


def _fold_tables(w0_ref, w1_ref, w2_ref, pw_ref, pb_ref, t_ref):
    for i, w_ref in enumerate((w0_ref, w1_ref, w2_ref)):
        acc = lax.dot_general(
            w_ref[...],
            pw_ref[:, i * HALF:(i + 1) * HALF],
            (((1,), (1,)), ((), ())),
            preferred_element_type=jnp.float32,
        )
        if i == 0:
            acc = acc + pb_ref[...]
        t_ref[i] = acc


def _sc_body(coords, tbl, out, cb, tloc, obufs, osems):
    cid = lax.axis_index("c")
    sid = lax.axis_index("s")
    wid = sid * 2 + cid
    rbase = wid * (VPW * 8)

    pltpu.sync_copy(tbl, tloc)
    pltpu.sync_copy(coords.at[pl.ds(wid * (VPW * 3), VPW * 3)], cb)

    nblk = D // 16
    lane = lax.iota(jnp.int32, 16)
    lane3 = lane * 3
    colb = [lane + (c16 * 16) for c16 in range(nblk)]

    def make_combine(slot, chunk):
        ob = obufs[slot]
        civ = lane3 + chunk * (CH * 3)
        xv = plsc.load_gather(cb, [civ])
        yv = plsc.load_gather(cb, [civ + 1])
        zv = plsc.load_gather(cb, [civ + 2])
        xe = xv * 512
        ye = yv * 512 + 50 * D
        ze = zv * 512 + 100 * D
        rowvecs = (xe, xe + D, ye, ye + D, ze, ze + D)

        def rows_of(v):
            spl = jnp.zeros((16,), jnp.int32) + v
            return tuple(rv[spl] for rv in rowvecs)

        def load_blk(rows, c16):
            col = colb[c16]
            return tuple(plsc.load_gather(tloc, [r + col]) for r in rows)

        def store_blk(r0, c16, regs):
            a0, a1, b0, b1, c0, c1 = regs
            sl = pl.ds(c16 * 16, 16)
            s00 = a0 + b0
            s01 = a0 + b1
            s10 = a1 + b0
            s11 = a1 + b1
            ob[r0 + 0, sl] = s00 + c0
            ob[r0 + 1, sl] = s00 + c1
            ob[r0 + 2, sl] = s01 + c0
            ob[r0 + 3, sl] = s01 + c1
            ob[r0 + 4, sl] = s10 + c0
            ob[r0 + 5, sl] = s10 + c1
            ob[r0 + 6, sl] = s11 + c0
            ob[r0 + 7, sl] = s11 + c1

        def body(vv, carry):
            va = vv * 2
            vb2 = va + 1
            rows_a = rows_of(va)
            rows_b = rows_of(vb2)
            regs_a = load_blk(rows_a, 0)
            regs_b = load_blk(rows_b, 0)
            for c16 in range(nblk):
                cur_a, cur_b = regs_a, regs_b
                if c16 + 1 < nblk:
                    regs_a = load_blk(rows_a, c16 + 1)
                    regs_b = load_blk(rows_b, c16 + 1)
                store_blk(va * 8, c16, cur_a)
                store_blk(vb2 * 8, c16, cur_b)
            return carry

        lax.fori_loop(0, CH // 2, body, 0)

    def loop_body(i, carry):
        for j in range(2):
            chunk = 2 * i + j

            @pl.when(i > 0)
            def _():
                pltpu.make_async_copy(
                    obufs[j], out.at[pl.ds(rbase, ROWS)], osems[j]
                ).wait()

            make_combine(j, chunk)
            pltpu.async_copy(
                obufs[j], out.at[pl.ds(rbase + chunk * ROWS, ROWS)], osems[j]
            )
        return carry

    lax.fori_loop(0, NCH // 2, loop_body, 0)

    for j in range(2):
        pltpu.make_async_copy(
            obufs[j], out.at[pl.ds(rbase, ROWS)], osems[j]
        ).wait()


def _sc_kernel_fn():
    mesh = plsc.VectorSubcoreMesh(
        core_axis_name="c", subcore_axis_name="s", num_cores=2, num_subcores=16
    )
    scratch = (
        [pltpu.VMEM((VPW * 3,), jnp.int32)]
        + [pltpu.VMEM((150 * D,), jnp.float32)]
        + [pltpu.VMEM((ROWS, D), jnp.float32) for _ in range(2)]
        + [pltpu.SemaphoreType.DMA for _ in range(2)]
    )

    def body(coords, tbl, out, *s):
        cb = s[0]
        tloc = s[1]
        obufs = s[2:4]
        osems = s[4:6]
        _sc_body(coords, tbl, out, cb, tloc, obufs, osems)

    return pl.kernel(
        body,
        out_type=jax.ShapeDtypeStruct((NQ * 8, D), jnp.float32),
        mesh=mesh,
        scratch_types=scratch,
        compiler_params=pltpu.CompilerParams(needs_layout_passes=False),
    )


def kernel(voxel_coordinates, w0, w1, w2, proj_w, proj_b):
    tables = pl.pallas_call(
        _fold_tables,
        out_shape=jax.ShapeDtypeStruct((3, 50, D), jnp.float32),
    )(w0, w1, w2, proj_w, proj_b.reshape(1, D))
    tbl = tables.reshape(150 * D)

    out = _sc_kernel_fn()(voxel_coordinates.reshape(NQ * 3), tbl)
    return out.reshape(1, NQ * 8, D)


# speedup vs baseline: 1.1280x; 1.1280x over previous
import functools

import jax
import jax.numpy as jnp
from jax import lax
from jax.experimental import pallas as pl
from jax.experimental.pallas import tpu as pltpu
from jax.experimental.pallas import tpu_sc as plsc

HALF = 128
D = 256
NQ = 16384
NW = 32
VPW = NQ // NW
CH = 16
NCH = VPW // CH
ROWS = CH * 8


def _fold_tables(w0_ref, w1_ref, w2_ref, pw_ref, pb_ref, t_ref):
    for i, w_ref in enumerate((w0_ref, w1_ref, w2_ref)):
        acc = lax.dot_general(
            w_ref[...],
            pw_ref[:, i * HALF:(i + 1) * HALF],
            (((1,), (1,)), ((), ())),
            preferred_element_type=jnp.float32,
        )
        if i == 0:
            acc = acc + pb_ref[...]
        t_ref[i] = acc


def _sc_body(xs, ys, zs, tbl, out, xb, yb, zb, tloc, obufs, osems):
    cid = lax.axis_index("c")
    sid = lax.axis_index("s")
    wid = sid * 2 + cid
    vbase = wid * VPW
    rbase = wid * (VPW * 8)

    start = [
        pltpu.async_copy(tbl, tloc, osems[0]),
        pltpu.async_copy(xs.at[pl.ds(vbase, VPW)], xb, osems[0]),
        pltpu.async_copy(ys.at[pl.ds(vbase, VPW)], yb, osems[0]),
        pltpu.async_copy(zs.at[pl.ds(vbase, VPW)], zb, osems[0]),
    ]
    for h in start:
        h.wait()

    nblk = D // 16
    lane = lax.iota(jnp.int32, 16)
    colb = [lane + (c16 * 16) for c16 in range(nblk)]

    def make_combine(slot, chunk):
        ob = obufs[slot]
        off = chunk * CH
        xv = xb[pl.ds(off, CH)]
        yv = yb[pl.ds(off, CH)]
        zv = zb[pl.ds(off, CH)]
        xe = xv * 512
        ye = yv * 512 + 50 * D
        ze = zv * 512 + 100 * D
        rowvecs = (xe, xe + D, ye, ye + D, ze, ze + D)

        def rows_of(v):
            spl = jnp.zeros((16,), jnp.int32) + v
            return tuple(rv[spl] for rv in rowvecs)

        def load_blk(rows, c16):
            col = colb[c16]
            return tuple(plsc.load_gather(tloc, [r + col]) for r in rows)

        def store_blk(r0, c16, regs):
            a0, a1, b0, b1, c0, c1 = regs
            sl = pl.ds(c16 * 16, 16)
            s00 = a0 + b0
            s01 = a0 + b1
            s10 = a1 + b0
            s11 = a1 + b1
            ob[r0 + 0, sl] = s00 + c0
            ob[r0 + 1, sl] = s00 + c1
            ob[r0 + 2, sl] = s01 + c0
            ob[r0 + 3, sl] = s01 + c1
            ob[r0 + 4, sl] = s10 + c0
            ob[r0 + 5, sl] = s10 + c1
            ob[r0 + 6, sl] = s11 + c0
            ob[r0 + 7, sl] = s11 + c1

        def body(vv, carry):
            va = vv * 2
            vb2 = va + 1
            rows_a = rows_of(va)
            rows_b = rows_of(vb2)
            regs_a = load_blk(rows_a, 0)
            regs_b = load_blk(rows_b, 0)
            for c16 in range(nblk):
                cur_a, cur_b = regs_a, regs_b
                if c16 + 1 < nblk:
                    regs_a = load_blk(rows_a, c16 + 1)
                    regs_b = load_blk(rows_b, c16 + 1)
                store_blk(va * 8, c16, cur_a)
                store_blk(vb2 * 8, c16, cur_b)
            return carry

        lax.fori_loop(0, CH // 2, body, 0)

    def loop_body(i, carry):
        for j in range(2):
            chunk = 2 * i + j

            @pl.when(i > 0)
            def _():
                pltpu.make_async_copy(
                    obufs[j], out.at[pl.ds(rbase, ROWS)], osems[j]
                ).wait()

            make_combine(j, chunk)
            pltpu.async_copy(
                obufs[j], out.at[pl.ds(rbase + chunk * ROWS, ROWS)], osems[j]
            )
        return carry

    lax.fori_loop(0, NCH // 2, loop_body, 0)

    for j in range(2):
        pltpu.make_async_copy(
            obufs[j], out.at[pl.ds(rbase, ROWS)], osems[j]
        ).wait()


def _sc_kernel_fn():
    mesh = plsc.VectorSubcoreMesh(
        core_axis_name="c", subcore_axis_name="s", num_cores=2, num_subcores=16
    )
    scratch = (
        [pltpu.VMEM((VPW,), jnp.int32) for _ in range(3)]
        + [pltpu.VMEM((150 * D,), jnp.float32)]
        + [pltpu.VMEM((ROWS, D), jnp.float32) for _ in range(2)]
        + [pltpu.SemaphoreType.DMA for _ in range(2)]
    )

    def body(xs, ys, zs, tbl, out, *s):
        xb, yb, zb = s[0:3]
        tloc = s[3]
        obufs = s[4:6]
        osems = s[6:8]
        _sc_body(xs, ys, zs, tbl, out, xb, yb, zb, tloc, obufs, osems)

    return pl.kernel(
        body,
        out_type=jax.ShapeDtypeStruct((NQ * 8, D), jnp.float32),
        mesh=mesh,
        scratch_types=scratch,
        compiler_params=pltpu.CompilerParams(needs_layout_passes=False),
    )


def kernel(voxel_coordinates, w0, w1, w2, proj_w, proj_b):
    tables = pl.pallas_call(
        _fold_tables,
        out_shape=jax.ShapeDtypeStruct((3, 50, D), jnp.float32),
    )(w0, w1, w2, proj_w, proj_b.reshape(1, D))
    tbl = tables.reshape(150 * D)

    xs = voxel_coordinates[:, 0]
    ys = voxel_coordinates[:, 1]
    zs = voxel_coordinates[:, 2]

    out = _sc_kernel_fn()(xs, ys, zs, tbl)
    return out.reshape(1, NQ * 8, D)
